# baseline (speedup 1.0000x reference)
import jax
import jax.numpy as jnp
from jax.experimental import pallas as pl
from jax.experimental.pallas import tpu as pltpu

NUM_EXPERTS = 64
EMBED_DIM = 2048
NUM_TOKENS = 16384

BT = 512
KS = 4
KC = EMBED_DIM // KS


def _router_body(*refs):
    x_refs = refs[:KS]
    wt_ref, onehot_ref, pmax_ref, logits_ref = refs[KS:]
    wt = wt_ref[...]
    logits = jnp.dot(x_refs[0][...], wt[:KC], preferred_element_type=jnp.float32)
    for k in range(1, KS):
        logits += jnp.dot(x_refs[k][...], wt[k * KC:(k + 1) * KC],
                          preferred_element_type=jnp.float32)
    logits_ref[...] = logits
    m = jnp.max(logits, axis=1, keepdims=True)
    s = jnp.sum(jnp.exp(logits - m), axis=1, keepdims=True)
    pmax_ref[...] = 1.0 / s
    idx = jnp.argmax(logits, axis=1)
    iota = jax.lax.broadcasted_iota(jnp.int32, logits.shape, 1)
    onehot_ref[...] = (iota == idx[:, None]).astype(jnp.int32)


@jax.jit
def kernel(hidden_states, W):
    wt = W.T
    grid = (NUM_TOKENS // BT,)

    def xspec(k):
        return pl.BlockSpec((BT, KC), lambda i, k=k: (i, k))

    onehot, pmax, logits = pl.pallas_call(
        _router_body,
        grid=grid,
        in_specs=[xspec(k) for k in range(KS)]
        + [pl.BlockSpec((EMBED_DIM, NUM_EXPERTS), lambda i: (0, 0))],
        out_specs=[
            pl.BlockSpec((BT, NUM_EXPERTS), lambda i: (i, 0)),
            pl.BlockSpec((BT, 1), lambda i: (i, 0)),
            pl.BlockSpec((BT, NUM_EXPERTS), lambda i: (i, 0)),
        ],
        out_shape=[
            jax.ShapeDtypeStruct((NUM_TOKENS, NUM_EXPERTS), jnp.int32),
            jax.ShapeDtypeStruct((NUM_TOKENS, 1), jnp.float32),
            jax.ShapeDtypeStruct((NUM_TOKENS, NUM_EXPERTS), jnp.float32),
        ],
        compiler_params=pltpu.CompilerParams(
            dimension_semantics=("arbitrary",),
        ),
    )(*([hidden_states] * KS + [wt]))
    return (onehot, pmax, logits)


# speedup vs baseline: 1.1288x; 1.1288x over previous
import jax
import jax.numpy as jnp
from jax.experimental import pallas as pl
from jax.experimental.pallas import tpu as pltpu

NUM_EXPERTS = 64
EMBED_DIM = 2048
NUM_TOKENS = 16384

RS = 4
BT = 512
BIG = RS * BT


def _router_body(*refs):
    x_refs = refs[:RS]
    wt_ref, onehot_ref, pmax_ref, logits_ref = refs[RS:]
    wt = wt_ref[...]
    for r in range(RS):
        sl = pl.ds(r * BT, BT)
        logits = jnp.dot(x_refs[r][...], wt, preferred_element_type=jnp.float32)
        logits_ref[sl, :] = logits
        m = jnp.max(logits, axis=1, keepdims=True)
        s = jnp.sum(jnp.exp(logits - m), axis=1, keepdims=True)
        pmax_ref[sl, :] = 1.0 / s
        idx = jnp.argmax(logits, axis=1)
        iota = jax.lax.broadcasted_iota(jnp.int32, logits.shape, 1)
        onehot_ref[sl, :] = (iota == idx[:, None]).astype(jnp.int32)


@jax.jit
def kernel(hidden_states, W):
    wt = W.T
    grid = (NUM_TOKENS // BIG,)

    def xspec(r):
        return pl.BlockSpec((BT, EMBED_DIM), lambda i, r=r: (RS * i + r, 0))

    onehot, pmax, logits = pl.pallas_call(
        _router_body,
        grid=grid,
        in_specs=[xspec(r) for r in range(RS)]
        + [pl.BlockSpec((EMBED_DIM, NUM_EXPERTS), lambda i: (0, 0))],
        out_specs=[
            pl.BlockSpec((BIG, NUM_EXPERTS), lambda i: (i, 0)),
            pl.BlockSpec((BIG, 1), lambda i: (i, 0)),
            pl.BlockSpec((BIG, NUM_EXPERTS), lambda i: (i, 0)),
        ],
        out_shape=[
            jax.ShapeDtypeStruct((NUM_TOKENS, NUM_EXPERTS), jnp.int32),
            jax.ShapeDtypeStruct((NUM_TOKENS, 1), jnp.float32),
            jax.ShapeDtypeStruct((NUM_TOKENS, NUM_EXPERTS), jnp.float32),
        ],
        compiler_params=pltpu.CompilerParams(
            dimension_semantics=("arbitrary",),
        ),
    )(*([hidden_states] * RS + [wt]))
    return (onehot, pmax, logits)
